# baseline (speedup 1.0000x reference)
import functools

import jax
import jax.numpy as jnp
from jax import lax
from jax.experimental import pallas as pl
from jax.experimental.pallas import tpu as pltpu
from jax.experimental.pallas import tpu_sc as plsc

S = 7
B = 2
C = 20
CH = B * 5 + C
CELLS = S * S
WPS = CELLS * CH
N = 8192
NW = 32
SAMPLES_PER_W = N // NW
CS = 16
NCHUNKS = SAMPLES_PER_W // CS
CHUNK_WORDS = CS * WPS
GROUPS = CS * CELLS // 16
GSTRIDE = 16 * CH


def _sqrt16(x):
    i = lax.bitcast_convert_type(x, jnp.int32)
    i = jnp.int32(0x5F3759DF) - lax.shift_right_arithmetic(i, 1)
    y = lax.bitcast_convert_type(i, jnp.float32)
    for _ in range(3):
        y = y * (1.5 - 0.5 * x * y * y)
    return x * y


def _group_body(g, carry, pbuf, tbuf, lane30):
    acc_coord, acc_obj, acc_noobj, acc_cls = carry
    idx0 = lane30 + g * GSTRIDE

    def ld(buf, c):
        return plsc.load_gather(buf, [idx0 + c])

    tm = {}
    cls = None
    for c in range(10, 30):
        tv = ld(tbuf, c)
        if c in (14, 19, 24, 29):
            tm[c] = tv
        d = ld(pbuf, c) - tv
        d2 = d * d
        cls = d2 if cls is None else cls + d2

    t = [ld(tbuf, c) for c in range(10)]
    p = [ld(pbuf, c) for c in range(10)]

    mrest = jnp.maximum(jnp.maximum(tm[14], tm[19]),
                        jnp.maximum(tm[24], tm[29]))
    c4 = t[4] > 0.0
    c9r = jnp.maximum(t[9], mrest) > 0.0
    obj = jnp.logical_or(c4, c9r)
    use_b1 = jnp.logical_and(jnp.logical_not(c4), c9r)

    tbx = jnp.where(use_b1, t[5], t[0])
    tby = jnp.where(use_b1, t[6], t[1])
    tbw = jnp.where(use_b1, t[7], t[2])
    tbh = jnp.where(use_b1, t[8], t[3])

    thw = tbw * 0.5
    thh = tbh * 0.5
    tx1 = tbx - thw
    tx2 = tbx + thw
    ty1 = tby - thh
    ty2 = tby + thh
    ta = tbw * tbh

    def iou(px, py, pw, ph):
        phw = pw * 0.5
        phh = ph * 0.5
        ix1 = jnp.maximum(px - phw, tx1)
        ix2 = jnp.minimum(px + phw, tx2)
        iy1 = jnp.maximum(py - phh, ty1)
        iy2 = jnp.minimum(py + phh, ty2)
        iw = jnp.maximum(ix2 - ix1, 0.0)
        ih = jnp.maximum(iy2 - iy1, 0.0)
        inter = iw * ih
        union = pw * ph + ta - inter
        return inter / (union + 1e-6)

    iou0 = iou(p[0], p[1], p[2], p[3])
    iou1 = iou(p[5], p[6], p[7], p[8])
    best1 = iou1 > iou0
    best_iou = jnp.maximum(iou0, iou1)

    sx = jnp.where(best1, p[5], p[0])
    sy = jnp.where(best1, p[6], p[1])
    sw = jnp.where(best1, p[7], p[2])
    sh = jnp.where(best1, p[8], p[3])
    sc = jnp.where(best1, p[9], p[4])

    zero = jnp.zeros((16,), jnp.float32)

    dx = sx - tbx
    dy = sy - tby
    xy = dx * dx + dy * dy

    pwc = jnp.maximum(sw, 1e-6)
    phc = jnp.maximum(sh, 1e-6)
    twc = jnp.maximum(tbw, 1e-6)
    thc = jnp.maximum(tbh, 1e-6)
    wh = (pwc + twc + phc + thc) - 2.0 * (_sqrt16(pwc * twc) + _sqrt16(phc * thc))

    acc_coord = acc_coord + jnp.where(obj, xy + wh, zero)

    dconf = sc - best_iou
    acc_obj = acc_obj + jnp.where(obj, dconf * dconf, zero)

    acc_noobj = acc_noobj + (p[4] * p[4] + p[9] * p[9]
                             - jnp.where(obj, sc * sc, zero))

    acc_cls = acc_cls + jnp.where(obj, cls, zero)

    return acc_coord, acc_obj, acc_noobj, acc_cls


def _make_kernel(nchunks):
    mesh = plsc.VectorSubcoreMesh(core_axis_name="c", subcore_axis_name="s")

    @functools.partial(
        pl.kernel,
        mesh=mesh,
        out_type=jax.ShapeDtypeStruct((NW, 4, 16), jnp.float32),
        scratch_types=[
            pltpu.VMEM((CHUNK_WORDS,), jnp.float32),
            pltpu.VMEM((CHUNK_WORDS,), jnp.float32),
            pltpu.VMEM((CHUNK_WORDS,), jnp.float32),
            pltpu.VMEM((CHUNK_WORDS,), jnp.float32),
            pltpu.VMEM((4, 16), jnp.float32),
            pltpu.SemaphoreType.DMA,
            pltpu.SemaphoreType.DMA,
            pltpu.SemaphoreType.DMA,
            pltpu.SemaphoreType.DMA,
        ],
        compiler_params=pltpu.CompilerParams(needs_layout_passes=False),
    )
    def yolo_loss(pred_hbm, targ_hbm, out_hbm, pbuf0, tbuf0, pbuf1, tbuf1,
                  accbuf, sp0, st0, sp1, st1):
        wid = lax.axis_index("s") * 2 + lax.axis_index("c")
        lane30 = lax.iota(jnp.int32, 16) * CH
        zero = jnp.zeros((16,), jnp.float32)
        bufs = ((pbuf0, tbuf0, sp0, st0), (pbuf1, tbuf1, sp1, st1))

        def start(k, slot):
            pb, tb, sp, st = bufs[slot]
            row = wid * nchunks + k
            pltpu.async_copy(pred_hbm.at[row], pb, sp)
            pltpu.async_copy(targ_hbm.at[row], tb, st)

        def wait(k, slot):
            pb, tb, sp, st = bufs[slot]
            row = wid * nchunks + k
            pltpu.make_async_copy(pred_hbm.at[row], pb, sp).wait()
            pltpu.make_async_copy(targ_hbm.at[row], tb, st).wait()

        def compute(slot, carry):
            pb, tb, _, _ = bufs[slot]

            @plsc.parallel_loop(0, GROUPS, unroll=2, carry=carry)
            def accs(g, c):
                return _group_body(g, c, pb, tb, lane30)

            return accs

        start(0, 0)

        def pair_body(m, carry):
            k0 = 2 * m
            wait(k0, 0)
            start(k0 + 1, 1)
            carry = compute(0, carry)
            wait(k0 + 1, 1)

            @pl.when(m < nchunks // 2 - 1)
            def _():
                start(k0 + 2, 0)

            return compute(1, carry)

        accs = lax.fori_loop(0, nchunks // 2, pair_body,
                             (zero, zero, zero, zero))
        accbuf[0, :] = accs[0]
        accbuf[1, :] = accs[1]
        accbuf[2, :] = accs[2]
        accbuf[3, :] = accs[3]
        pltpu.sync_copy(accbuf, out_hbm.at[wid])

    return yolo_loss


NSLICE = 2
SLICE_CHUNKS = N // NSLICE // NW // CS
_yolo_slice = _make_kernel(SLICE_CHUNKS)


@jax.jit
def kernel(predictions, targets):
    n = predictions.shape[0]
    ns = n // NSLICE
    rows = NW * SLICE_CHUNKS
    sums = jnp.zeros((4,), jnp.float32)
    for i in range(NSLICE):
        pred2d = lax.slice_in_dim(predictions, i * ns, (i + 1) * ns
                                  ).reshape(rows, CHUNK_WORDS)
        targ2d = lax.slice_in_dim(targets, i * ns, (i + 1) * ns
                                  ).reshape(rows, CHUNK_WORDS)
        parts = _yolo_slice(pred2d, targ2d)
        sums = sums + jnp.sum(parts, axis=(0, 2))
    coord = 5.0 * sums[0] / n
    conf_obj = sums[1] / n
    conf_noobj = 0.5 * sums[2] / n
    cls = sums[3] / n
    total = coord + conf_obj + conf_noobj + cls
    return (total, coord, conf_obj, conf_noobj, cls)


# speedup vs baseline: 1.2131x; 1.2131x over previous
import functools

import jax
import jax.numpy as jnp
from jax import lax
from jax.experimental import pallas as pl
from jax.experimental.pallas import tpu as pltpu
from jax.experimental.pallas import tpu_sc as plsc

S = 7
B = 2
C = 20
CH = B * 5 + C
CELLS = S * S
WPS = CELLS * CH
N = 8192
NW = 32
RW = 128
TOTAL_ROWS = N * WPS // RW
ROWS_PER_W = TOTAL_ROWS // NW
CHUNK_ROWS = 120
CHUNK_WORDS = CHUNK_ROWS * RW
TOTAL_CHUNKS = TOTAL_ROWS // CHUNK_ROWS
GROUPS = CHUNK_WORDS // CH // 16
GSTRIDE = 16 * CH


def _sqrt16(x):
    i = lax.bitcast_convert_type(x, jnp.int32)
    i = jnp.int32(0x5F3759DF) - lax.shift_right_arithmetic(i, 1)
    y = lax.bitcast_convert_type(i, jnp.float32)
    for _ in range(3):
        y = y * (1.5 - 0.5 * x * y * y)
    return x * y


def _group_body(g, carry, pbuf, tbuf, lane30):
    acc_coord, acc_obj, acc_noobj, acc_cls = carry
    idx0 = lane30 + g * GSTRIDE

    def ld(buf, c):
        idx = idx0 + c
        return plsc.load_gather(
            buf, [lax.shift_right_logical(idx, 7), lax.bitwise_and(idx, 127)])

    tm = {}
    cls = None
    for c in range(10, 30):
        tv = ld(tbuf, c)
        if c in (14, 19, 24, 29):
            tm[c] = tv
        d = ld(pbuf, c) - tv
        d2 = d * d
        cls = d2 if cls is None else cls + d2

    t = [ld(tbuf, c) for c in range(10)]
    p = [ld(pbuf, c) for c in range(10)]

    mrest = jnp.maximum(jnp.maximum(tm[14], tm[19]),
                        jnp.maximum(tm[24], tm[29]))
    c4 = t[4] > 0.0
    c9r = jnp.maximum(t[9], mrest) > 0.0
    obj = jnp.logical_or(c4, c9r)
    use_b1 = jnp.logical_and(jnp.logical_not(c4), c9r)

    tbx = jnp.where(use_b1, t[5], t[0])
    tby = jnp.where(use_b1, t[6], t[1])
    tbw = jnp.where(use_b1, t[7], t[2])
    tbh = jnp.where(use_b1, t[8], t[3])

    thw = tbw * 0.5
    thh = tbh * 0.5
    tx1 = tbx - thw
    tx2 = tbx + thw
    ty1 = tby - thh
    ty2 = tby + thh
    ta = tbw * tbh

    def iou(px, py, pw, ph):
        phw = pw * 0.5
        phh = ph * 0.5
        ix1 = jnp.maximum(px - phw, tx1)
        ix2 = jnp.minimum(px + phw, tx2)
        iy1 = jnp.maximum(py - phh, ty1)
        iy2 = jnp.minimum(py + phh, ty2)
        iw = jnp.maximum(ix2 - ix1, 0.0)
        ih = jnp.maximum(iy2 - iy1, 0.0)
        inter = iw * ih
        union = pw * ph + ta - inter
        return inter / (union + 1e-6)

    iou0 = iou(p[0], p[1], p[2], p[3])
    iou1 = iou(p[5], p[6], p[7], p[8])
    best1 = iou1 > iou0
    best_iou = jnp.maximum(iou0, iou1)

    sx = jnp.where(best1, p[5], p[0])
    sy = jnp.where(best1, p[6], p[1])
    sw = jnp.where(best1, p[7], p[2])
    sh = jnp.where(best1, p[8], p[3])
    sc = jnp.where(best1, p[9], p[4])

    zero = jnp.zeros((16,), jnp.float32)

    dx = sx - tbx
    dy = sy - tby
    xy = dx * dx + dy * dy

    pwc = jnp.maximum(sw, 1e-6)
    phc = jnp.maximum(sh, 1e-6)
    twc = jnp.maximum(tbw, 1e-6)
    thc = jnp.maximum(tbh, 1e-6)
    wh = (pwc + twc + phc + thc) - 2.0 * (_sqrt16(pwc * twc) + _sqrt16(phc * thc))

    acc_coord = acc_coord + jnp.where(obj, xy + wh, zero)

    dconf = sc - best_iou
    acc_obj = acc_obj + jnp.where(obj, dconf * dconf, zero)

    acc_noobj = acc_noobj + (p[4] * p[4] + p[9] * p[9]
                             - jnp.where(obj, sc * sc, zero))

    acc_cls = acc_cls + jnp.where(obj, cls, zero)

    return acc_coord, acc_obj, acc_noobj, acc_cls


def _make_kernel():
    mesh = plsc.VectorSubcoreMesh(core_axis_name="c", subcore_axis_name="s")

    @functools.partial(
        pl.kernel,
        mesh=mesh,
        out_type=jax.ShapeDtypeStruct((NW, 4, 16), jnp.float32),
        scratch_types=[
            pltpu.VMEM((CHUNK_ROWS, RW), jnp.float32),
            pltpu.VMEM((CHUNK_ROWS, RW), jnp.float32),
            pltpu.VMEM((CHUNK_ROWS, RW), jnp.float32),
            pltpu.VMEM((CHUNK_ROWS, RW), jnp.float32),
            pltpu.VMEM((4, 16), jnp.float32),
            pltpu.SemaphoreType.DMA,
            pltpu.SemaphoreType.DMA,
            pltpu.SemaphoreType.DMA,
            pltpu.SemaphoreType.DMA,
        ],
        compiler_params=pltpu.CompilerParams(needs_layout_passes=False),
    )
    def yolo_loss(pred_hbm, targ_hbm, out_hbm, pbuf0, tbuf0, pbuf1, tbuf1,
                  accbuf, sp0, st0, sp1, st1):
        wid = lax.axis_index("s") * 2 + lax.axis_index("c")
        lane30 = lax.iota(jnp.int32, 16) * CH
        zero = jnp.zeros((16,), jnp.float32)
        bufs = ((pbuf0, tbuf0, sp0, st0), (pbuf1, tbuf1, sp1, st1))

        u0 = lax.shift_right_logical(wid * (TOTAL_CHUNKS // 2), 4)
        u1 = lax.shift_right_logical((wid + 1) * (TOTAL_CHUNKS // 2), 4)

        def start(k, slot):
            pb, tb, sp, st = bufs[slot]
            r0 = k * CHUNK_ROWS
            pltpu.async_copy(pred_hbm.at[pl.ds(r0, CHUNK_ROWS)], pb, sp)
            pltpu.async_copy(targ_hbm.at[pl.ds(r0, CHUNK_ROWS)], tb, st)

        def wait(k, slot):
            pb, tb, sp, st = bufs[slot]
            r0 = k * CHUNK_ROWS
            pltpu.make_async_copy(
                pred_hbm.at[pl.ds(r0, CHUNK_ROWS)], pb, sp).wait()
            pltpu.make_async_copy(
                targ_hbm.at[pl.ds(r0, CHUNK_ROWS)], tb, st).wait()

        def compute(slot, carry):
            pb, tb, _, _ = bufs[slot]

            @plsc.parallel_loop(0, GROUPS, unroll=2, carry=carry)
            def accs(g, c):
                return _group_body(g, c, pb, tb, lane30)

            return accs

        def chunk_step(k, carry, slot):
            wait(k, slot)

            @pl.when(k + 1 < u1)
            def _():
                start(k + 1, 1 - slot)

            return compute(slot, carry)

        def body(k, carry):
            return lax.cond(
                lax.bitwise_and(k - u0, 1) == 0,
                lambda c: chunk_step(k, c, 0),
                lambda c: chunk_step(k, c, 1),
                carry)

        start(u0, 0)
        accs = lax.fori_loop(u0, u1, body, (zero, zero, zero, zero))
        accbuf[0, :] = accs[0]
        accbuf[1, :] = accs[1]
        accbuf[2, :] = accs[2]
        accbuf[3, :] = accs[3]
        pltpu.sync_copy(accbuf, out_hbm.at[wid])

    return yolo_loss


_yolo_loss = _make_kernel()


@jax.jit
def kernel(predictions, targets):
    n = predictions.shape[0]
    pred2d = predictions.reshape(TOTAL_ROWS, RW)
    targ2d = targets.reshape(TOTAL_ROWS, RW)
    parts = _yolo_loss(pred2d, targ2d)
    sums = jnp.sum(parts, axis=(0, 2))
    coord = 5.0 * sums[0] / n
    conf_obj = sums[1] / n
    conf_noobj = 0.5 * sums[2] / n
    cls = sums[3] / n
    total = coord + conf_obj + conf_noobj + cls
    return (total, coord, conf_obj, conf_noobj, cls)


# speedup vs baseline: 1.7727x; 1.4614x over previous
import functools

import jax
import jax.numpy as jnp
from jax import lax
from jax.experimental import pallas as pl
from jax.experimental.pallas import tpu as pltpu
from jax.experimental.pallas import tpu_sc as plsc

S = 7
B = 2
C = 20
CH = B * 5 + C
CELLS = S * S
WPS = CELLS * CH
N = 8192
NW = 32
SAMPLES_PER_W = N // NW
CS = 16
NCHUNKS = SAMPLES_PER_W // CS
CHUNK_WORDS = CS * WPS
GROUPS = CS * CELLS // 16
GSTRIDE = 16 * CH


def _sqrt16(x):
    i = lax.bitcast_convert_type(x, jnp.int32)
    i = jnp.int32(0x5F3759DF) - lax.shift_right_arithmetic(i, 1)
    y = lax.bitcast_convert_type(i, jnp.float32)
    for _ in range(3):
        y = y * (1.5 - 0.5 * x * y * y)
    return x * y


def _group_body(g, carry, pbuf, tbuf, lane30):
    acc_coord, acc_obj, acc_noobj, acc_cls = carry
    idx0 = lane30 + g * GSTRIDE

    def ld(buf, c):
        return plsc.load_gather(buf, [idx0 + c])

    tm = {}
    cls = None
    for c in range(10, 30):
        tv = ld(tbuf, c)
        if c in (14, 19, 24, 29):
            tm[c] = tv
        d = ld(pbuf, c) - tv
        d2 = d * d
        cls = d2 if cls is None else cls + d2

    t = [ld(tbuf, c) for c in range(10)]
    p = [ld(pbuf, c) for c in range(10)]

    mrest = jnp.maximum(jnp.maximum(tm[14], tm[19]),
                        jnp.maximum(tm[24], tm[29]))
    c4 = t[4] > 0.0
    c9r = jnp.maximum(t[9], mrest) > 0.0
    obj = jnp.logical_or(c4, c9r)
    use_b1 = jnp.logical_and(jnp.logical_not(c4), c9r)

    tbx = jnp.where(use_b1, t[5], t[0])
    tby = jnp.where(use_b1, t[6], t[1])
    tbw = jnp.where(use_b1, t[7], t[2])
    tbh = jnp.where(use_b1, t[8], t[3])

    thw = tbw * 0.5
    thh = tbh * 0.5
    tx1 = tbx - thw
    tx2 = tbx + thw
    ty1 = tby - thh
    ty2 = tby + thh
    ta = tbw * tbh

    def iou(px, py, pw, ph):
        phw = pw * 0.5
        phh = ph * 0.5
        ix1 = jnp.maximum(px - phw, tx1)
        ix2 = jnp.minimum(px + phw, tx2)
        iy1 = jnp.maximum(py - phh, ty1)
        iy2 = jnp.minimum(py + phh, ty2)
        iw = jnp.maximum(ix2 - ix1, 0.0)
        ih = jnp.maximum(iy2 - iy1, 0.0)
        inter = iw * ih
        union = pw * ph + ta - inter
        return inter / (union + 1e-6)

    iou0 = iou(p[0], p[1], p[2], p[3])
    iou1 = iou(p[5], p[6], p[7], p[8])
    best1 = iou1 > iou0
    best_iou = jnp.maximum(iou0, iou1)

    sx = jnp.where(best1, p[5], p[0])
    sy = jnp.where(best1, p[6], p[1])
    sw = jnp.where(best1, p[7], p[2])
    sh = jnp.where(best1, p[8], p[3])
    sc = jnp.where(best1, p[9], p[4])

    zero = jnp.zeros((16,), jnp.float32)

    dx = sx - tbx
    dy = sy - tby
    xy = dx * dx + dy * dy

    pwc = jnp.maximum(sw, 1e-6)
    phc = jnp.maximum(sh, 1e-6)
    twc = jnp.maximum(tbw, 1e-6)
    thc = jnp.maximum(tbh, 1e-6)
    wh = (pwc + twc + phc + thc) - 2.0 * (_sqrt16(pwc * twc) + _sqrt16(phc * thc))

    acc_coord = acc_coord + jnp.where(obj, xy + wh, zero)

    dconf = sc - best_iou
    acc_obj = acc_obj + jnp.where(obj, dconf * dconf, zero)

    acc_noobj = acc_noobj + (p[4] * p[4] + p[9] * p[9]
                             - jnp.where(obj, sc * sc, zero))

    acc_cls = acc_cls + jnp.where(obj, cls, zero)

    return acc_coord, acc_obj, acc_noobj, acc_cls


def _make_kernel():
    mesh = plsc.VectorSubcoreMesh(core_axis_name="c", subcore_axis_name="s")

    @functools.partial(
        pl.kernel,
        mesh=mesh,
        out_type=jax.ShapeDtypeStruct((NW, 4, 16), jnp.float32),
        scratch_types=[
            pltpu.VMEM((CHUNK_WORDS,), jnp.float32),
            pltpu.VMEM((CHUNK_WORDS,), jnp.float32),
            pltpu.VMEM((CHUNK_WORDS,), jnp.float32),
            pltpu.VMEM((CHUNK_WORDS,), jnp.float32),
            pltpu.VMEM((4, 16), jnp.float32),
            pltpu.SemaphoreType.DMA,
            pltpu.SemaphoreType.DMA,
            pltpu.SemaphoreType.DMA,
            pltpu.SemaphoreType.DMA,
        ],
        compiler_params=pltpu.CompilerParams(needs_layout_passes=False),
    )
    def yolo_loss(pred_hbm, targ_hbm, out_hbm, pbuf0, tbuf0, pbuf1, tbuf1,
                  accbuf, sp0, st0, sp1, st1):
        wid = lax.axis_index("s") * 2 + lax.axis_index("c")
        lane30 = lax.iota(jnp.int32, 16) * CH
        zero = jnp.zeros((16,), jnp.float32)
        bufs = ((pbuf0, tbuf0, sp0, st0), (pbuf1, tbuf1, sp1, st1))

        def start(k, slot):
            pb, tb, sp, st = bufs[slot]
            row = wid * NCHUNKS + k
            pltpu.async_copy(pred_hbm.at[row], pb, sp)
            pltpu.async_copy(targ_hbm.at[row], tb, st)

        def wait(k, slot):
            pb, tb, sp, st = bufs[slot]
            row = wid * NCHUNKS + k
            pltpu.make_async_copy(pred_hbm.at[row], pb, sp).wait()
            pltpu.make_async_copy(targ_hbm.at[row], tb, st).wait()

        def compute(slot, carry):
            pb, tb, _, _ = bufs[slot]

            @plsc.parallel_loop(0, GROUPS, unroll=4, carry=carry)
            def accs(g, c):
                return _group_body(g, c, pb, tb, lane30)

            return accs

        start(0, 0)

        def pair_body(m, carry):
            k0 = 2 * m
            wait(k0, 0)
            start(k0 + 1, 1)
            carry = compute(0, carry)
            wait(k0 + 1, 1)

            @pl.when(m < NCHUNKS // 2 - 1)
            def _():
                start(k0 + 2, 0)

            return compute(1, carry)

        accs = lax.fori_loop(0, NCHUNKS // 2, pair_body,
                             (zero, zero, zero, zero))
        accbuf[0, :] = accs[0]
        accbuf[1, :] = accs[1]
        accbuf[2, :] = accs[2]
        accbuf[3, :] = accs[3]
        pltpu.sync_copy(accbuf, out_hbm.at[wid])

    return yolo_loss


_yolo_loss = _make_kernel()


@jax.jit
def kernel(predictions, targets):
    n = predictions.shape[0]
    pred2d = predictions.reshape(NW * NCHUNKS, CHUNK_WORDS)
    targ2d = targets.reshape(NW * NCHUNKS, CHUNK_WORDS)
    parts = _yolo_loss(pred2d, targ2d)
    sums = jnp.sum(parts, axis=(0, 2))
    coord = 5.0 * sums[0] / n
    conf_obj = sums[1] / n
    conf_noobj = 0.5 * sums[2] / n
    cls = sums[3] / n
    total = coord + conf_obj + conf_noobj + cls
    return (total, coord, conf_obj, conf_noobj, cls)


# speedup vs baseline: 1.7789x; 1.0035x over previous
import functools

import jax
import jax.numpy as jnp
from jax import lax
from jax.experimental import pallas as pl
from jax.experimental.pallas import tpu as pltpu
from jax.experimental.pallas import tpu_sc as plsc

S = 7
B = 2
C = 20
CH = B * 5 + C
CELLS = S * S
WPS = CELLS * CH
N = 8192
NW = 32
SAMPLES_PER_W = N // NW
CS = 16
NCHUNKS = SAMPLES_PER_W // CS
CHUNK_WORDS = CS * WPS
GROUPS = CS * CELLS // 16
GSTRIDE = 16 * CH


def _sqrt16(x):
    i = lax.bitcast_convert_type(x, jnp.int32)
    i = jnp.int32(0x5F3759DF) - lax.shift_right_arithmetic(i, 1)
    y = lax.bitcast_convert_type(i, jnp.float32)
    for _ in range(3):
        y = y * (1.5 - 0.5 * x * y * y)
    return x * y


def _group_body(g, carry, pbuf, tbuf, lane30):
    acc_coord, acc_obj, acc_noobj, acc_cls = carry
    idx0 = lane30 + g * GSTRIDE

    def ld(buf, c):
        return plsc.load_gather(buf, [idx0 + c])

    tm = {}
    cls = None
    for c in range(10, 30):
        tv = ld(tbuf, c)
        if c in (14, 19, 24, 29):
            tm[c] = tv
        d = ld(pbuf, c) - tv
        d2 = d * d
        cls = d2 if cls is None else cls + d2

    t = [ld(tbuf, c) for c in range(10)]
    p = [ld(pbuf, c) for c in range(10)]

    mrest = jnp.maximum(jnp.maximum(tm[14], tm[19]),
                        jnp.maximum(tm[24], tm[29]))
    c4 = t[4] > 0.0
    c9r = jnp.maximum(t[9], mrest) > 0.0
    obj = jnp.logical_or(c4, c9r)
    use_b1 = jnp.logical_and(jnp.logical_not(c4), c9r)

    tbx = jnp.where(use_b1, t[5], t[0])
    tby = jnp.where(use_b1, t[6], t[1])
    tbw = jnp.where(use_b1, t[7], t[2])
    tbh = jnp.where(use_b1, t[8], t[3])

    thw = tbw * 0.5
    thh = tbh * 0.5
    tx1 = tbx - thw
    tx2 = tbx + thw
    ty1 = tby - thh
    ty2 = tby + thh
    ta = tbw * tbh

    def iou(px, py, pw, ph):
        phw = pw * 0.5
        phh = ph * 0.5
        ix1 = jnp.maximum(px - phw, tx1)
        ix2 = jnp.minimum(px + phw, tx2)
        iy1 = jnp.maximum(py - phh, ty1)
        iy2 = jnp.minimum(py + phh, ty2)
        iw = jnp.maximum(ix2 - ix1, 0.0)
        ih = jnp.maximum(iy2 - iy1, 0.0)
        inter = iw * ih
        union = pw * ph + ta - inter
        return inter / (union + 1e-6)

    iou0 = iou(p[0], p[1], p[2], p[3])
    iou1 = iou(p[5], p[6], p[7], p[8])
    best1 = iou1 > iou0
    best_iou = jnp.maximum(iou0, iou1)

    sx = jnp.where(best1, p[5], p[0])
    sy = jnp.where(best1, p[6], p[1])
    sw = jnp.where(best1, p[7], p[2])
    sh = jnp.where(best1, p[8], p[3])
    sc = jnp.where(best1, p[9], p[4])

    zero = jnp.zeros((16,), jnp.float32)

    dx = sx - tbx
    dy = sy - tby
    xy = dx * dx + dy * dy

    pwc = jnp.maximum(sw, 1e-6)
    phc = jnp.maximum(sh, 1e-6)
    twc = jnp.maximum(tbw, 1e-6)
    thc = jnp.maximum(tbh, 1e-6)
    wh = (pwc + twc + phc + thc) - 2.0 * (_sqrt16(pwc * twc) + _sqrt16(phc * thc))

    acc_coord = acc_coord + jnp.where(obj, xy + wh, zero)

    dconf = sc - best_iou
    acc_obj = acc_obj + jnp.where(obj, dconf * dconf, zero)

    acc_noobj = acc_noobj + (p[4] * p[4] + p[9] * p[9]
                             - jnp.where(obj, sc * sc, zero))

    acc_cls = acc_cls + jnp.where(obj, cls, zero)

    return acc_coord, acc_obj, acc_noobj, acc_cls


def _make_kernel():
    mesh = plsc.VectorSubcoreMesh(core_axis_name="c", subcore_axis_name="s")

    @functools.partial(
        pl.kernel,
        mesh=mesh,
        out_type=jax.ShapeDtypeStruct((NW, 4, 16), jnp.float32),
        scratch_types=[
            pltpu.VMEM((CHUNK_WORDS,), jnp.float32),
            pltpu.VMEM((CHUNK_WORDS,), jnp.float32),
            pltpu.VMEM((CHUNK_WORDS,), jnp.float32),
            pltpu.VMEM((CHUNK_WORDS,), jnp.float32),
            pltpu.VMEM((4, 16), jnp.float32),
            pltpu.SemaphoreType.DMA,
            pltpu.SemaphoreType.DMA,
            pltpu.SemaphoreType.DMA,
            pltpu.SemaphoreType.DMA,
        ],
        compiler_params=pltpu.CompilerParams(needs_layout_passes=False),
    )
    def yolo_loss(pred_hbm, targ_hbm, out_hbm, pbuf0, tbuf0, pbuf1, tbuf1,
                  accbuf, sp0, st0, sp1, st1):
        wid = lax.axis_index("s") * 2 + lax.axis_index("c")
        lane30 = lax.iota(jnp.int32, 16) * CH
        zero = jnp.zeros((16,), jnp.float32)
        bufs = ((pbuf0, tbuf0, sp0, st0), (pbuf1, tbuf1, sp1, st1))

        def start(k, slot):
            pb, tb, sp, st = bufs[slot]
            row = wid * NCHUNKS + k
            pltpu.async_copy(pred_hbm.at[row], pb, sp)
            pltpu.async_copy(targ_hbm.at[row], tb, st)

        def wait(k, slot):
            pb, tb, sp, st = bufs[slot]
            row = wid * NCHUNKS + k
            pltpu.make_async_copy(pred_hbm.at[row], pb, sp).wait()
            pltpu.make_async_copy(targ_hbm.at[row], tb, st).wait()

        def compute(slot, carry):
            pb, tb, _, _ = bufs[slot]

            @plsc.parallel_loop(0, GROUPS, unroll=7, carry=carry)
            def accs(g, c):
                return _group_body(g, c, pb, tb, lane30)

            return accs

        start(0, 0)

        def pair_body(m, carry):
            k0 = 2 * m
            wait(k0, 0)
            start(k0 + 1, 1)
            carry = compute(0, carry)
            wait(k0 + 1, 1)

            @pl.when(m < NCHUNKS // 2 - 1)
            def _():
                start(k0 + 2, 0)

            return compute(1, carry)

        accs = lax.fori_loop(0, NCHUNKS // 2, pair_body,
                             (zero, zero, zero, zero))
        accbuf[0, :] = accs[0]
        accbuf[1, :] = accs[1]
        accbuf[2, :] = accs[2]
        accbuf[3, :] = accs[3]
        pltpu.sync_copy(accbuf, out_hbm.at[wid])

    return yolo_loss


_yolo_loss = _make_kernel()


@jax.jit
def kernel(predictions, targets):
    n = predictions.shape[0]
    pred2d = predictions.reshape(NW * NCHUNKS, CHUNK_WORDS)
    targ2d = targets.reshape(NW * NCHUNKS, CHUNK_WORDS)
    parts = _yolo_loss(pred2d, targ2d)
    sums = jnp.sum(parts, axis=(0, 2))
    coord = 5.0 * sums[0] / n
    conf_obj = sums[1] / n
    conf_noobj = 0.5 * sums[2] / n
    cls = sums[3] / n
    total = coord + conf_obj + conf_noobj + cls
    return (total, coord, conf_obj, conf_noobj, cls)
